# baseline (speedup 1.0000x reference)
import jax
import jax.numpy as jnp
from jax import lax
from jax.experimental import pallas as pl
from jax.experimental.pallas import tpu as pltpu

_B, _C_IN, _T = 8, 96, 1024
_C_OUT, _K = 32, 512
_BB = 4
_TT = _BB * _T


def _vq_body(x_ref, w_ref, b_ref, cb_ref, quant_ref, norms_ref):
    zs = [
        lax.dot_general(
            x_ref[i], w_ref[...], (((0,), (1,)), ((), ())),
            preferred_element_type=jnp.float32)
        for i in range(_BB)
    ]
    z = jnp.concatenate(zs, axis=0) + b_ref[...]

    zz = jnp.sum(z * z, axis=1, keepdims=True)
    cb = cb_ref[...]
    cn = jnp.sum(cb * cb, axis=1)
    s = lax.dot_general(
        z, cb, (((1,), (1,)), ((), ())),
        preferred_element_type=jnp.float32)

    d2 = (zz - 2.0 * s) + cn[None, :]

    m = jnp.min(d2, axis=1, keepdims=True)
    kio = lax.broadcasted_iota(jnp.int32, (_TT, _K), 1)
    idx = jnp.min(jnp.where(d2 == m, kio, _K), axis=1, keepdims=True)
    onehot = (kio == idx).astype(jnp.float32)

    qT = lax.dot_general(
        cb, onehot, (((0,), (1,)), ((), ())),
        preferred_element_type=jnp.float32)

    for i in range(_BB):
        quant_ref[i] = qT[:, i * _T:(i + 1) * _T]
        norms_ref[i] = jnp.broadcast_to(m[i * _T:(i + 1) * _T], (_T, 2))


def kernel(x, W, b, codebook):
    b_col = b[None, :]
    grid = (_B // _BB,)
    quant, norms = pl.pallas_call(
        _vq_body,
        grid=grid,
        in_specs=[
            pl.BlockSpec((_BB, _C_IN, _T), lambda i: (i, 0, 0)),
            pl.BlockSpec((_C_OUT, _C_IN), lambda i: (0, 0)),
            pl.BlockSpec((1, _C_OUT), lambda i: (0, 0)),
            pl.BlockSpec((_K, _C_OUT), lambda i: (0, 0)),
        ],
        out_specs=[
            pl.BlockSpec((_BB, _C_OUT, _T), lambda i: (i, 0, 0)),
            pl.BlockSpec((_BB, _T, 2), lambda i: (i, 0, 0)),
        ],
        out_shape=[
            jax.ShapeDtypeStruct((_B, _C_OUT, _T), jnp.float32),
            jax.ShapeDtypeStruct((_B, _T, 2), jnp.float32),
        ],
        compiler_params=pltpu.CompilerParams(
            dimension_semantics=("parallel",)),
    )(x, W, b_col, codebook)
    return quant, norms


# speedup vs baseline: 1.0249x; 1.0249x over previous
import jax
import jax.numpy as jnp
from jax import lax
from jax.experimental import pallas as pl
from jax.experimental.pallas import tpu as pltpu

_B, _C_IN, _T = 8, 96, 1024
_C_OUT, _K = 32, 512
_BB = 2
_TT = _BB * _T


def _vq_body(x_ref, w_ref, b_ref, cb_ref, quant_ref, norms_ref):
    zs = [
        lax.dot_general(
            x_ref[i], w_ref[...], (((0,), (1,)), ((), ())),
            preferred_element_type=jnp.float32)
        for i in range(_BB)
    ]
    z = jnp.concatenate(zs, axis=0) + b_ref[...]

    zz = jnp.sum(z * z, axis=1, keepdims=True)
    cb = cb_ref[...]
    cn = jnp.sum(cb * cb, axis=1)
    s2 = lax.dot_general(
        z, cb + cb, (((1,), (1,)), ((), ())),
        preferred_element_type=jnp.float32)

    d2 = (zz - s2) + cn[None, :]

    m = jnp.min(d2, axis=1, keepdims=True)
    kio = lax.broadcasted_iota(jnp.int32, (_TT, _K), 1)
    idx = jnp.min(jnp.where(d2 == m, kio, _K), axis=1, keepdims=True)
    onehot = (kio == idx).astype(jnp.bfloat16)

    qT = lax.dot_general(
        cb.astype(jnp.bfloat16), onehot, (((0,), (1,)), ((), ())),
        preferred_element_type=jnp.float32)

    for i in range(_BB):
        quant_ref[i] = qT[:, i * _T:(i + 1) * _T]
        norms_ref[i] = jnp.broadcast_to(m[i * _T:(i + 1) * _T], (_T, 2))


def kernel(x, W, b, codebook):
    b_col = b[None, :]
    grid = (_B // _BB,)
    quant, norms = pl.pallas_call(
        _vq_body,
        grid=grid,
        in_specs=[
            pl.BlockSpec((_BB, _C_IN, _T), lambda i: (i, 0, 0)),
            pl.BlockSpec((_C_OUT, _C_IN), lambda i: (0, 0)),
            pl.BlockSpec((1, _C_OUT), lambda i: (0, 0)),
            pl.BlockSpec((_K, _C_OUT), lambda i: (0, 0)),
        ],
        out_specs=[
            pl.BlockSpec((_BB, _C_OUT, _T), lambda i: (i, 0, 0)),
            pl.BlockSpec((_BB, _T, 2), lambda i: (i, 0, 0)),
        ],
        out_shape=[
            jax.ShapeDtypeStruct((_B, _C_OUT, _T), jnp.float32),
            jax.ShapeDtypeStruct((_B, _T, 2), jnp.float32),
        ],
        compiler_params=pltpu.CompilerParams(
            dimension_semantics=("parallel",)),
    )(x, W, b_col, codebook)
    return quant, norms
